# Initial kernel scaffold; baseline (speedup 1.0000x reference)
import jax
import jax.numpy as jnp
from jax.experimental import pallas as pl


def kernel(x, pos_embedding_weight):
    raise NotImplementedError("write your pallas kernel here")


# speedup vs baseline: 1.7191x; 1.7191x over previous
import jax
import jax.numpy as jnp
from jax.experimental import pallas as pl


_SEQ_BLOCK = 256


def _add_kernel(x_ref, w_ref, o_ref):
    o_ref[...] = x_ref[...] + w_ref[...][None, :, :]


def kernel(x, pos_embedding_weight):
    batch, seq_len, emb_dim = x.shape
    sb = _SEQ_BLOCK
    grid = (seq_len // sb,)
    return pl.pallas_call(
        _add_kernel,
        grid=grid,
        in_specs=[
            pl.BlockSpec((batch, sb, emb_dim), lambda s: (0, s, 0)),
            pl.BlockSpec((sb, emb_dim), lambda s: (s, 0)),
        ],
        out_specs=pl.BlockSpec((batch, sb, emb_dim), lambda s: (0, s, 0)),
        out_shape=jax.ShapeDtypeStruct(x.shape, x.dtype),
    )(x, pos_embedding_weight[:seq_len])


# speedup vs baseline: 1.7242x; 1.0030x over previous
import jax
import jax.numpy as jnp
from jax.experimental import pallas as pl


_SEQ_BLOCK = 512


def _add_kernel(x_ref, w_ref, o_ref):
    o_ref[...] = x_ref[...] + w_ref[...][None, :, :]


def kernel(x, pos_embedding_weight):
    batch, seq_len, emb_dim = x.shape
    sb = _SEQ_BLOCK
    grid = (seq_len // sb,)
    return pl.pallas_call(
        _add_kernel,
        grid=grid,
        in_specs=[
            pl.BlockSpec((batch, sb, emb_dim), lambda s: (0, s, 0)),
            pl.BlockSpec((sb, emb_dim), lambda s: (s, 0)),
        ],
        out_specs=pl.BlockSpec((batch, sb, emb_dim), lambda s: (0, s, 0)),
        out_shape=jax.ShapeDtypeStruct(x.shape, x.dtype),
    )(x, pos_embedding_weight[:seq_len])
